# Initial kernel scaffold; baseline (speedup 1.0000x reference)
import jax
import jax.numpy as jnp
from jax.experimental import pallas as pl


def kernel(preds, targets):
    raise NotImplementedError("write your pallas kernel here")


# speedup vs baseline: 5.9928x; 5.9928x over previous
import jax
import jax.numpy as jnp
from jax import lax
from jax.experimental import pallas as pl
from jax.experimental.pallas import tpu as pltpu
from jax.experimental.pallas import tpu_sc as plsc

B = 4
N = 20000
M = 20
NPAD = 20480
NCHUNK = 8
CH = NPAD // NCHUNK
L = 16
NVEC = CH // L
GTG = 4
BIG_IDX = 1 << 30


def _onehot_merge(lane, pos, scalar, vec):
    return jnp.where(lane == pos, scalar, vec)


def _sc_kernel_body(comp_hbm, tgt_hbm, out_hbm,
                    px, py, pw, ph, cf, px2, py2, pA,
                    tg, loc_f, loc_i, mrg_f, mrg_i, outrow,
                    shf, shi):
    c = lax.axis_index("c")
    s = lax.axis_index("s")
    b = c * 2 + s // NCHUNK
    chunk = s % NCHUNK
    base = chunk * CH

    for k, ref in ((0, px), (1, py), (2, pw), (3, ph), (4, cf)):
        off = (k * B + b) * NPAD + base
        pltpu.sync_copy(comp_hbm.at[pl.ds(off, CH)], ref)
    pltpu.sync_copy(tgt_hbm.at[pl.ds(b * 128, 128)], tg)

    def _pre(v, _):
        sl = pl.ds(v * L, L)
        px2[sl] = px[sl] + pw[sl]
        py2[sl] = py[sl] + ph[sl]
        pA[sl] = pw[sl] * ph[sl]
        return 0
    lax.fori_loop(0, NVEC, _pre, 0)

    lane = lax.broadcasted_iota(jnp.int32, (L,), 0)

    tgv = {}
    for comp_i in range(4):
        tgv[comp_i] = (tg[pl.ds(comp_i * 32, L)], tg[pl.ds(comp_i * 32 + L, L)])

    def _gt_scalar(comp_i, m):
        return tgv[comp_i][m // L][m % L]

    locm = [jnp.full((L,), -1.0, jnp.float32) for _ in range(2)]
    loci = [jnp.full((L,), base, jnp.int32) for _ in range(2)]

    for g in range(M // GTG):
        gts = []
        for mi in range(GTG):
            m = g * GTG + mi
            gx = _gt_scalar(0, m)
            gy = _gt_scalar(1, m)
            gw = _gt_scalar(2, m)
            gh = _gt_scalar(3, m)
            gts.append((gx, gy, gx + gw, gy + gh, gw * gh))

        def _scan(v, carry):
            idxv = base + v * L + lane
            sl = pl.ds(v * L, L)
            vx, vy, vx2, vy2, vA = px[sl], py[sl], px2[sl], py2[sl], pA[sl]
            out = []
            for mi in range(GTG):
                gx, gy, gx2, gy2, gA = gts[mi]
                mcur, icur = carry[mi]
                xa = jnp.maximum(vx, gx)
                ya = jnp.maximum(vy, gy)
                xb = jnp.minimum(vx2, gx2)
                yb = jnp.minimum(vy2, gy2)
                inter = jnp.maximum(xb - xa, 0.0) * jnp.maximum(yb - ya, 0.0)
                union = vA + (gA - inter)
                upos = union > 0.0
                iou = jnp.where(upos, inter / jnp.where(upos, union, 1.0), 0.0)
                upd = iou > mcur
                out.append((jnp.where(upd, iou, mcur),
                            jnp.where(upd, idxv, icur)))
            return tuple(out)

        init = tuple((jnp.full((L,), -1.0, jnp.float32),
                      jnp.full((L,), base, jnp.int32)) for _ in range(GTG))
        res = lax.fori_loop(0, NVEC, _scan, init)

        for mi in range(GTG):
            m = g * GTG + mi
            mvec, ivec = res[mi]
            mval = jnp.max(mvec)
            best = jnp.min(jnp.where(mvec == mval, ivec, BIG_IDX))
            locm[m // L] = _onehot_merge(lane, m % L, mval, locm[m // L])
            loci[m // L] = _onehot_merge(lane, m % L, best, loci[m // L])

    loc_f[pl.ds(0, L)] = locm[0]
    loc_f[pl.ds(L, L)] = locm[1]
    loc_i[pl.ds(0, L)] = loci[0]
    loc_i[pl.ds(L, L)] = loci[1]

    for half in range(2):
        rel = loci[half] - base
        for fi, ref in ((1, px), (2, py), (3, pw), (4, ph), (5, cf)):
            loc_f[pl.ds((fi * 2 + half) * L, L)] = plsc.load_gather(ref, [rel])

    pltpu.sync_copy(loc_f, shf.at[pl.ds(s * 384, 384)])
    pltpu.sync_copy(loc_i, shi.at[pl.ds(s * 32, 32)])
    plsc.subcore_barrier()

    @pl.when(s % NCHUNK == 0)
    def _merge():
        pltpu.sync_copy(shf.at[pl.ds(s * 384, NCHUNK * 384)], mrg_f)
        pltpu.sync_copy(shi.at[pl.ds(s * 32, NCHUNK * 32)], mrg_i)

        gmax = [jnp.full((L,), -1.0, jnp.float32) for _ in range(2)]
        gidx = [jnp.zeros((L,), jnp.int32) for _ in range(2)]
        gbox = [[jnp.zeros((L,), jnp.float32) for _ in range(2)]
                for _ in range(5)]
        for ci in range(NCHUNK):
            for half in range(2):
                cmax = mrg_f[pl.ds(ci * 384 + half * L, L)]
                cidx = mrg_i[pl.ds(ci * 32 + half * L, L)]
                upd = cmax > gmax[half]
                gmax[half] = jnp.where(upd, cmax, gmax[half])
                gidx[half] = jnp.where(upd, cidx, gidx[half])
                for fi in range(5):
                    cbox = mrg_f[pl.ds(ci * 384 + ((fi + 1) * 2 + half) * L, L)]
                    gbox[fi][half] = jnp.where(upd, cbox, gbox[fi][half])

        hit = [gmax[h] > 0.5 for h in range(2)]
        hiti = [hit[h].astype(jnp.int32) for h in range(2)]

        mpos = [lane, lane + L]
        dup = [jnp.zeros((L,), jnp.bool_) for _ in range(2)]
        for mp in range(M):
            jm = gidx[mp // L][mp % L]
            hm = hiti[mp // L][mp % L] > 0
            for h in range(2):
                clash = hm & (gidx[h] == jm) & (mpos[h] > mp)
                dup[h] = dup[h] | clash
        valid = [hit[h] & (~dup[h]) for h in range(2)]
        key = [jnp.where(valid[h], gidx[h], BIG_IDX) for h in range(2)]

        rank = [jnp.zeros((L,), jnp.int32) for _ in range(2)]
        for mp in range(M):
            km = key[mp // L][mp % L]
            for h in range(2):
                rank[h] = rank[h] + (key[h] > km).astype(jnp.int32)

        nval = jnp.sum(valid[0].astype(jnp.int32)) + \
            jnp.sum(valid[1].astype(jnp.int32))
        sx = jnp.sum(jnp.where(valid[0], gbox[4][0], 0.0)) + \
            jnp.sum(jnp.where(valid[1], gbox[4][1], 0.0))

        bbox = jnp.zeros((L,), jnp.float32)
        for h in range(2):
            acc = jnp.zeros((L,), jnp.float32)
            for fi in range(4):
                tcomp = plsc.load_gather(tg, [fi * 32 + rank[h]])
                d = gbox[fi][h] - tcomp
                acc = acc + d * d
            bbox = bbox + jnp.where(valid[h], acc, 0.0)
        bb = jnp.sum(bbox)

        out_v = jnp.where(lane == 0, nval.astype(jnp.float32),
                          jnp.where(lane == 1, sx,
                                    jnp.where(lane == 2, bb, 0.0)))
        outrow[pl.ds(0, L)] = out_v
        pltpu.sync_copy(outrow, out_hbm.at[pl.ds(b * L, L)])


def _tc_softplus_body(x_ref, o_ref):
    x = x_ref[0]
    g = jnp.maximum(x, 0.0) + jnp.log1p(jnp.exp(-jnp.abs(x)))
    o_ref[0] = jnp.full((8, 128), jnp.sum(g), jnp.float32)


@jax.jit
def kernel(preds, targets):
    f32 = jnp.float32
    comp = jnp.transpose(preds, (2, 0, 1))
    padc = jnp.concatenate([
        jnp.full((2, B, NPAD - N), 2.0, f32),
        jnp.zeros((2, B, NPAD - N), f32),
        jnp.full((1, B, NPAD - N), -1e30, f32),
    ], axis=0)
    comp = jnp.concatenate([comp, padc], axis=2)
    comp_flat = comp.reshape(5 * B * NPAD)
    tgt = jnp.transpose(targets, (0, 2, 1))
    tgt_flat = jnp.pad(tgt, ((0, 0), (0, 0), (0, 32 - M))).reshape(B * 4 * 32)

    mesh = plsc.VectorSubcoreMesh(core_axis_name="c", subcore_axis_name="s")
    sc_call = pl.kernel(
        _sc_kernel_body,
        out_type=jax.ShapeDtypeStruct((B * L,), f32),
        mesh=mesh,
        compiler_params=pltpu.CompilerParams(needs_layout_passes=False),
        scratch_types=[
            pltpu.VMEM((CH,), f32),
            pltpu.VMEM((CH,), f32),
            pltpu.VMEM((CH,), f32),
            pltpu.VMEM((CH,), f32),
            pltpu.VMEM((CH,), f32),
            pltpu.VMEM((CH,), f32),
            pltpu.VMEM((CH,), f32),
            pltpu.VMEM((CH,), f32),
            pltpu.VMEM((128,), f32),
            pltpu.VMEM((384,), f32),
            pltpu.VMEM((32,), jnp.int32),
            pltpu.VMEM((NCHUNK * 384,), f32),
            pltpu.VMEM((NCHUNK * 32,), jnp.int32),
            pltpu.VMEM((L,), f32),
            pltpu.VMEM_SHARED((16 * 384,), f32),
            pltpu.VMEM_SHARED((16 * 32,), jnp.int32),
        ],
    )
    sc_out = sc_call(comp_flat, tgt_flat).reshape(B, L)

    cf3 = comp[4].reshape(B, NPAD // 128, 128)
    s0_call = pl.pallas_call(
        _tc_softplus_body,
        out_shape=jax.ShapeDtypeStruct((B, 8, 128), f32),
        grid=(B,),
        in_specs=[pl.BlockSpec((1, NPAD // 128, 128), lambda i: (i, 0, 0))],
        out_specs=pl.BlockSpec((1, 8, 128), lambda i: (i, 0, 0)),
    )
    s0 = s0_call(cf3)[:, 0, 0]

    n = sc_out[:, 0]
    sx = sc_out[:, 1]
    bb = sc_out[:, 2]
    conf_loss = (s0 - sx) / N
    bbox_loss = bb / (jnp.maximum(n, 1.0) * 4.0)
    per_batch = jnp.where(n > 0, bbox_loss + conf_loss, 0.0)
    return jnp.asarray(jnp.mean(per_batch), f32)
